# baseline (speedup 1.0000x reference)
import jax
import jax.numpy as jnp
import numpy as np
from jax import lax
from jax.experimental import pallas as pl
from jax.experimental.pallas import tpu as pltpu
from jax.experimental.pallas import tpu_sc as plsc

NC = 2
NS = 16
CHUNK = 128
NBUF = 4


def _sc_spmm(x, pack, vals, *, scaled, nbuf, interpret=False):
    n, d = x.shape
    nchunks, npk, chunk = pack.shape
    assert d % 16 == 0 and npk == 2 and (vals is not None) == scaled
    assert chunk % 16 == 0 and chunk <= 128
    bf16 = x.dtype == jnp.bfloat16
    assert not bf16 or d % 32 == 0
    n_pad = -(-n // (NS * 8)) * NS * 8
    rps = n_pad // NS
    q, r = divmod(nchunks, NC * NS)
    mesh = plsc.VectorSubcoreMesh(core_axis_name="c", subcore_axis_name="s",
                                  num_cores=NC, num_subcores=NS)

    def body(x_hbm, pack_hbm, *rest):
        it = iter(rest)
        vals_hbm = next(it) if scaled else None
        zeros_hbm = next(it)
        out_hbm = next(it)
        acc = next(it)
        idxs = [[next(it) for _ in range(nbuf)] for _ in range(2)]
        vals_v = [[next(it) for _ in range(nbuf)] for _ in range(2)] if scaled else None
        bufs16 = [next(it) for _ in range(nbuf)] if bf16 else None
        bufs = [next(it) for _ in range(nbuf)]
        semi = [[next(it) for _ in range(nbuf)] for _ in range(2)]
        semg = [next(it) for _ in range(nbuf)]
        sems = [next(it) for _ in range(nbuf)]
        assert next(it, None) is None

        c = lax.axis_index("c")
        s = lax.axis_index("s")
        w = c * NS + s
        r0 = s * rps
        pltpu.sync_copy(zeros_hbm.at[pl.ds(r0, rps)], acc.at[pl.ds(r0, rps)])
        plsc.subcore_barrier()

        base = q * w + jnp.minimum(w, r)
        count = q + (w < r).astype(jnp.int32)

        def fire_idx(p, b, ch):
            pltpu.async_copy(pack_hbm.at[base + ch], idxs[p][b], semi[p][b])
            if scaled:
                pltpu.async_copy(vals_hbm.at[base + ch], vals_v[p][b], semi[p][b])

        def drain_idx(p, b, ch):
            pltpu.make_async_copy(pack_hbm.at[base + ch], idxs[p][b], semi[p][b]).wait()
            if scaled:
                pltpu.make_async_copy(vals_hbm.at[base + ch], vals_v[p][b], semi[p][b]).wait()

        gbufs = bufs16 if bf16 else bufs

        def fire_gather(p, b):
            pltpu.async_copy(x_hbm.at[idxs[p][b].at[1]], gbufs[b], semg[b])

        def drain_gather(p, b):
            pltpu.make_async_copy(x_hbm.at[idxs[p][b].at[1]], gbufs[b], semg[b]).wait()

        def fire_scatter(p, b):
            pltpu.async_copy(bufs[b], acc.at[idxs[p][b].at[0]], sems[b], add=True)

        def drain_scatter(b):
            pltpu.make_async_copy(bufs[b], acc.at[idxs[0][b].at[0]], sems[b]).wait()

        def scale(p, b):
            def scale_group(g, cc):
                v16 = vals_v[p][b][pl.ds(g * 16, 16)] if scaled else None
                for j in range(16):
                    e = g * 16 + j
                    v = v16[j] if scaled else None
                    if bf16:
                        for k in range(d // 32):
                            x32 = bufs16[b][e, pl.ds(k * 32, 32)]
                            lo, hi = plsc.unpack(
                                x32, format=plsc.PackFormat.INTERLEAVED)
                            if scaled:
                                lo = lo * v
                                hi = hi * v
                            bufs[b][e, pl.ds(k * 32, 16)] = lo
                            bufs[b][e, pl.ds(k * 32 + 16, 16)] = hi
                    else:
                        for k in range(d // 16):
                            sl = pl.ds(k * 16, 16)
                            bufs[b][e, sl] = bufs[b][e, sl] * v
                return cc

            lax.fori_loop(0, chunk // 16, scale_group, 0)

        for b in range(nbuf):
            @pl.when(b < count)
            def _(b=b):
                fire_idx(0, b, b)

        def run_group(gg, p):
            ch0 = gg * nbuf
            for b in range(nbuf):
                ch = ch0 + b

                @pl.when(ch < count)
                def _(b=b, ch=ch):
                    drain_idx(p, b, ch)

                    @pl.when(ch >= nbuf)
                    def _():
                        drain_scatter(b)

                    fire_gather(p, b)

            for b in range(nbuf):
                chn = ch0 + nbuf + b

                @pl.when(chn < count)
                def _(b=b, chn=chn):
                    fire_idx(1 - p, b, chn)

            for b in range(nbuf):
                ch = ch0 + b

                @pl.when(ch < count)
                def _(b=b, ch=ch):
                    drain_gather(p, b)
                    if scaled or bf16:
                        scale(p, b)
                    fire_scatter(p, b)

        def outer(g, carry):
            run_group(2 * g, 0)
            run_group(2 * g + 1, 1)
            return carry

        lax.fori_loop(0, (count + 2 * nbuf - 1) // (2 * nbuf), outer, 0)

        for b in range(nbuf):
            @pl.when(b < count)
            def _(b=b):
                drain_scatter(b)

        plsc.subcore_barrier()
        pltpu.sync_copy(acc.at[pl.ds(r0, rps)], out_hbm.at[c, pl.ds(r0, rps)])

    out_type = jax.ShapeDtypeStruct((NC, n_pad, d), jnp.float32)
    scratch = [pltpu.VMEM_SHARED((n_pad, d), jnp.float32)]
    scratch += [pltpu.VMEM((npk, chunk), jnp.int32) for _ in range(2 * nbuf)]
    if scaled:
        scratch += [pltpu.VMEM((chunk,), jnp.float32) for _ in range(2 * nbuf)]
    if bf16:
        scratch += [pltpu.VMEM((chunk, d), jnp.bfloat16) for _ in range(nbuf)]
    scratch += [pltpu.VMEM((chunk, d), jnp.float32) for _ in range(nbuf)]
    scratch += [pltpu.SemaphoreType.DMA for _ in range(4 * nbuf)]

    kfn = pl.kernel(
        body,
        out_type=out_type,
        mesh=mesh,
        scratch_types=scratch,
        compiler_params=pltpu.CompilerParams(use_tc_tiling_on_sc=False, needs_layout_passes=False),
        interpret=interpret,
    )
    args = [x, pack]
    if scaled:
        args.append(vals)
    args.append(jnp.zeros((n_pad, d), jnp.float32))
    return kfn(*args)


def _interleave_perm(d):
    p = np.zeros(d, dtype=np.int32)
    for k in range(d // 32):
        for j in range(16):
            p[32 * k + 2 * j] = 32 * k + j
            p[32 * k + 2 * j + 1] = 32 * k + 16 + j
    return p


def _perm_matrix(d):
    perm = _interleave_perm(d)
    m = np.zeros((d, d), dtype=np.float32)
    m[perm, np.arange(d)] = 1.0
    return jnp.asarray(m)


def _pack_edges(rows, cols, chunk):
    return jnp.concatenate(
        [rows.reshape(-1, 1, chunk), cols.reshape(-1, 1, chunk)], axis=1)


def _block_rows(n):
    for cand in (2000, 1000, 500, 200, 104, 80, 40, 16, 8):
        if n % cand == 0:
            return cand
    return n


def _tc_k1(feat, feat_a, w1p, *, interpret=False):
    n, din = feat.shape
    dout = w1p.shape[1]
    bn = _block_rows(n)

    def body(f_ref, fa_ref, w_ref, o_ref):
        w = w_ref[...]
        o_ref[:, :dout] = jnp.dot(
            f_ref[...], w, preferred_element_type=jnp.float32).astype(jnp.bfloat16)
        o_ref[:, dout:] = jnp.dot(
            fa_ref[...], w, preferred_element_type=jnp.float32).astype(jnp.bfloat16)

    return pl.pallas_call(
        body,
        grid=(n // bn,),
        in_specs=[pl.BlockSpec((bn, din), lambda i: (i, 0)),
                  pl.BlockSpec((bn, din), lambda i: (i, 0)),
                  pl.BlockSpec((din, dout), lambda i: (0, 0))],
        out_specs=pl.BlockSpec((bn, 2 * dout), lambda i: (i, 0)),
        out_shape=jax.ShapeDtypeStruct((n, 2 * dout), jnp.bfloat16),
        interpret=interpret,
    )(feat, feat_a, w1p)


def _tc_k2(n, pa, dw1t, db1, dw2t, db2, pm, *, interpret=False):
    d2 = pa.shape[2]
    d = d2 // 2
    bn = _block_rows(n)

    def body(pa_ref, w1t_ref, b1_ref, w2t_ref, b2_ref, pm_ref,
             hid_ref, e64_ref, e128_ref, dec_ref, deca_ref):
        z = pa_ref[0] + pa_ref[1]
        hid_ref[...] = z[:, :d]
        em = jnp.maximum(z, 0.0)
        e128_ref[...] = jnp.dot(
            em, pm_ref[...], preferred_element_type=jnp.float32).astype(jnp.bfloat16)
        e1 = em[:, :d]
        e2 = em[:, d:]
        e64_ref[...] = e1
        w1t = w1t_ref[...]
        w2t = w2t_ref[...]
        b1 = b1_ref[...]
        b2 = b2_ref[...]
        y = jnp.maximum(jnp.dot(e1, w1t, preferred_element_type=jnp.float32) + b1, 0.0)
        dec_ref[...] = jnp.dot(y, w2t, preferred_element_type=jnp.float32) + b2
        ya = jnp.maximum(jnp.dot(e2, w1t, preferred_element_type=jnp.float32) + b1, 0.0)
        deca_ref[...] = jnp.dot(ya, w2t, preferred_element_type=jnp.float32) + b2

    return pl.pallas_call(
        body,
        grid=(n // bn,),
        in_specs=[pl.BlockSpec((NC, bn, d2), lambda i: (0, i, 0)),
                  pl.BlockSpec((d, d), lambda i: (0, 0)),
                  pl.BlockSpec((1, d), lambda i: (0, 0)),
                  pl.BlockSpec((d, d), lambda i: (0, 0)),
                  pl.BlockSpec((1, d), lambda i: (0, 0)),
                  pl.BlockSpec((d2, d2), lambda i: (0, 0))],
        out_specs=[pl.BlockSpec((bn, d), lambda i: (i, 0)),
                   pl.BlockSpec((bn, d), lambda i: (i, 0)),
                   pl.BlockSpec((bn, d2), lambda i: (i, 0)),
                   pl.BlockSpec((bn, d), lambda i: (i, 0)),
                   pl.BlockSpec((bn, d), lambda i: (i, 0))],
        out_shape=[jax.ShapeDtypeStruct((n, d), jnp.float32),
                   jax.ShapeDtypeStruct((n, d), jnp.float32),
                   jax.ShapeDtypeStruct((n, d2), jnp.bfloat16),
                   jax.ShapeDtypeStruct((n, d), jnp.float32),
                   jax.ShapeDtypeStruct((n, d), jnp.float32)],
        interpret=interpret,
    )(pa, dw1t, db1, dw2t, db2, pm)


def _tc_k3(n, pb, pc, w2, *, interpret=False):
    d = pb.shape[2]
    d2 = pc.shape[2]
    din = w2.shape[1]
    bn = _block_rows(n)

    def body(pb_ref, pc_ref, w2_ref, h_ref, ret_ref, reta_ref):
        sagg = pb_ref[0] + pb_ref[1]
        h_ref[...] = jnp.dot(sagg, w2_ref[...], preferred_element_type=jnp.float32)
        g = pc_ref[0] + pc_ref[1]

        def norm_sig(x):
            nn = jnp.sqrt(jnp.sum(x * x, axis=1, keepdims=True))
            return jax.nn.sigmoid(x / jnp.maximum(nn, 1e-12))

        ret_ref[...] = norm_sig(g[:, :d])
        reta_ref[...] = norm_sig(g[:, d:])

    return pl.pallas_call(
        body,
        grid=(n // bn,),
        in_specs=[pl.BlockSpec((NC, bn, d), lambda i: (0, i, 0)),
                  pl.BlockSpec((NC, bn, d2), lambda i: (0, i, 0)),
                  pl.BlockSpec((d, din), lambda i: (0, 0))],
        out_specs=[pl.BlockSpec((bn, din), lambda i: (i, 0)),
                   pl.BlockSpec((bn, d), lambda i: (i, 0)),
                   pl.BlockSpec((bn, d), lambda i: (i, 0))],
        out_shape=[jax.ShapeDtypeStruct((n, din), jnp.float32),
                   jax.ShapeDtypeStruct((n, d), jnp.float32),
                   jax.ShapeDtypeStruct((n, d), jnp.float32)],
        interpret=interpret,
    )(pb, pc, w2)


def kernel(feat, feat_a, adj_indices, adj_values, neigh_indices, neigh_values,
           weight1, weight2, dec_w1, dec_b1, dec_w2, dec_b2):
    n = feat.shape[0]
    dout = weight1.shape[1]
    adj_i = adj_indices.astype(jnp.int32)
    nei_i = neigh_indices.astype(jnp.int32)
    pack_a80 = _pack_edges(adj_i[0], adj_i[1], 80)
    pack_a128 = _pack_edges(adj_i[0], adj_i[1], 128)
    pack_n80 = _pack_edges(nei_i[0], nei_i[1], 80)
    vals_a80 = adj_values.astype(jnp.float32).reshape(-1, 80)
    vals_a128 = adj_values.astype(jnp.float32).reshape(-1, 128)
    w1p = weight1[:, jnp.asarray(_interleave_perm(dout))]

    zc = _tc_k1(feat, feat_a, w1p)
    pa = _sc_spmm(zc, pack_a80, vals_a80, scaled=True, nbuf=3)
    hiden_emb, emb64, e128p, dec, dec_a = _tc_k2(
        n, pa, dec_w1.T, dec_b1.reshape(1, -1), dec_w2.T, dec_b2.reshape(1, -1),
        _perm_matrix(2 * dout))
    pb = _sc_spmm(emb64, pack_a128, vals_a128, scaled=True, nbuf=5)
    pc = _sc_spmm(e128p, pack_n80, None, scaled=False, nbuf=3)
    h, ret, ret_a = _tc_k3(n, pb, pc, weight2)
    return (hiden_emb, h, dec, dec_a, ret, ret_a)


# speedup vs baseline: 1.5498x; 1.5498x over previous
import jax
import jax.numpy as jnp
from jax import lax
from jax.experimental import pallas as pl
from jax.experimental.pallas import tpu as pltpu
from jax.experimental.pallas import tpu_sc as plsc

NC = 2
NS = 16
CHUNK = 128
NBUF = 4


def _sc_spmm(x, pack, vals, *, scaled, nbuf, interpret=False):
    n, d = x.shape
    nchunks, npk, chunk = pack.shape
    assert d % 16 == 0 and npk == 2 and (vals is not None) == scaled
    assert chunk % 16 == 0 and chunk <= 128
    n_pad = -(-n // (NS * 8)) * NS * 8
    rps = n_pad // NS
    q, r = divmod(nchunks, NC * NS)
    mesh = plsc.VectorSubcoreMesh(core_axis_name="c", subcore_axis_name="s",
                                  num_cores=NC, num_subcores=NS)

    def body(x_hbm, pack_hbm, *rest):
        it = iter(rest)
        vals_hbm = next(it) if scaled else None
        zeros_hbm = next(it)
        out_hbm = next(it)
        acc = next(it)
        idxs = [[next(it) for _ in range(nbuf)] for _ in range(2)]
        vals_v = [[next(it) for _ in range(nbuf)] for _ in range(2)] if scaled else None
        bufs = [next(it) for _ in range(nbuf)]
        semi = [[next(it) for _ in range(nbuf)] for _ in range(2)]
        semg = [next(it) for _ in range(nbuf)]
        sems = [next(it) for _ in range(nbuf)]
        assert next(it, None) is None

        c = lax.axis_index("c")
        s = lax.axis_index("s")
        w = c * NS + s
        r0 = s * rps
        pltpu.sync_copy(zeros_hbm.at[pl.ds(r0, rps)], acc.at[pl.ds(r0, rps)])
        plsc.subcore_barrier()

        base = q * w + jnp.minimum(w, r)
        count = q + (w < r).astype(jnp.int32)

        def fire_idx(p, b, ch):
            pltpu.async_copy(pack_hbm.at[base + ch], idxs[p][b], semi[p][b])
            if scaled:
                pltpu.async_copy(vals_hbm.at[base + ch], vals_v[p][b], semi[p][b])

        def drain_idx(p, b, ch):
            pltpu.make_async_copy(pack_hbm.at[base + ch], idxs[p][b], semi[p][b]).wait()
            if scaled:
                pltpu.make_async_copy(vals_hbm.at[base + ch], vals_v[p][b], semi[p][b]).wait()

        def fire_gather(p, b):
            pltpu.async_copy(x_hbm.at[idxs[p][b].at[1]], bufs[b], semg[b])

        def drain_gather(p, b):
            pltpu.make_async_copy(x_hbm.at[idxs[p][b].at[1]], bufs[b], semg[b]).wait()

        def fire_scatter(p, b):
            pltpu.async_copy(bufs[b], acc.at[idxs[p][b].at[0]], sems[b], add=True)

        def drain_scatter(b):
            pltpu.make_async_copy(bufs[b], acc.at[idxs[0][b].at[0]], sems[b]).wait()

        def scale(p, b):
            def scale_group(g, cc):
                v16 = vals_v[p][b][pl.ds(g * 16, 16)]
                for j in range(16):
                    v = v16[j]
                    for k in range(d // 16):
                        sl = pl.ds(k * 16, 16)
                        bufs[b][g * 16 + j, sl] = bufs[b][g * 16 + j, sl] * v
                return cc

            lax.fori_loop(0, chunk // 16, scale_group, 0)

        for b in range(nbuf):
            @pl.when(b < count)
            def _(b=b):
                fire_idx(0, b, b)

        def run_group(gg, p):
            ch0 = gg * nbuf
            for b in range(nbuf):
                ch = ch0 + b

                @pl.when(ch < count)
                def _(b=b, ch=ch):
                    drain_idx(p, b, ch)

                    @pl.when(ch >= nbuf)
                    def _():
                        drain_scatter(b)

                    fire_gather(p, b)

            for b in range(nbuf):
                chn = ch0 + nbuf + b

                @pl.when(chn < count)
                def _(b=b, chn=chn):
                    fire_idx(1 - p, b, chn)

            for b in range(nbuf):
                ch = ch0 + b

                @pl.when(ch < count)
                def _(b=b, ch=ch):
                    drain_gather(p, b)
                    if scaled:
                        scale(p, b)
                    fire_scatter(p, b)

        def outer(g, carry):
            run_group(2 * g, 0)
            run_group(2 * g + 1, 1)
            return carry

        lax.fori_loop(0, (count + 2 * nbuf - 1) // (2 * nbuf), outer, 0)

        for b in range(nbuf):
            @pl.when(b < count)
            def _(b=b):
                drain_scatter(b)

        plsc.subcore_barrier()
        pltpu.sync_copy(acc.at[pl.ds(r0, rps)], out_hbm.at[c, pl.ds(r0, rps)])

    out_type = jax.ShapeDtypeStruct((NC, n_pad, d), jnp.float32)
    scratch = [pltpu.VMEM_SHARED((n_pad, d), jnp.float32)]
    scratch += [pltpu.VMEM((npk, chunk), jnp.int32) for _ in range(2 * nbuf)]
    if scaled:
        scratch += [pltpu.VMEM((chunk,), jnp.float32) for _ in range(2 * nbuf)]
    scratch += [pltpu.VMEM((chunk, d), jnp.float32) for _ in range(nbuf)]
    scratch += [pltpu.SemaphoreType.DMA for _ in range(4 * nbuf)]

    kfn = pl.kernel(
        body,
        out_type=out_type,
        mesh=mesh,
        scratch_types=scratch,
        compiler_params=pltpu.CompilerParams(use_tc_tiling_on_sc=False),
        interpret=interpret,
    )
    args = [x, pack]
    if scaled:
        args.append(vals)
    args.append(jnp.zeros((n_pad, d), jnp.float32))
    return kfn(*args)


def _pack_edges(rows, cols, chunk):
    return jnp.concatenate(
        [rows.reshape(-1, 1, chunk), cols.reshape(-1, 1, chunk)], axis=1)


def _block_rows(n):
    for cand in (2000, 1000, 500, 200, 104, 80, 40, 16, 8):
        if n % cand == 0:
            return cand
    return n


def _tc_k1(feat, feat_a, w1, *, interpret=False):
    n, din = feat.shape
    dout = w1.shape[1]
    bn = _block_rows(n)

    def body(f_ref, fa_ref, w_ref, o_ref):
        w = w_ref[...]
        o_ref[:, :dout] = jnp.dot(f_ref[...], w, preferred_element_type=jnp.float32)
        o_ref[:, dout:] = jnp.dot(fa_ref[...], w, preferred_element_type=jnp.float32)

    return pl.pallas_call(
        body,
        grid=(n // bn,),
        in_specs=[pl.BlockSpec((bn, din), lambda i: (i, 0)),
                  pl.BlockSpec((bn, din), lambda i: (i, 0)),
                  pl.BlockSpec((din, dout), lambda i: (0, 0))],
        out_specs=pl.BlockSpec((bn, 2 * dout), lambda i: (i, 0)),
        out_shape=jax.ShapeDtypeStruct((n, 2 * dout), jnp.float32),
        interpret=interpret,
    )(feat, feat_a, w1)


def _tc_k2(n, pa, dw1t, db1, dw2t, db2, *, interpret=False):
    d2 = pa.shape[2]
    d = d2 // 2
    bn = _block_rows(n)

    def body(pa_ref, w1t_ref, b1_ref, w2t_ref, b2_ref,
             hid_ref, e64_ref, e128_ref, dec_ref, deca_ref):
        z = pa_ref[0] + pa_ref[1]
        hid_ref[...] = z[:, :d]
        em = jnp.maximum(z, 0.0)
        e128_ref[...] = em
        e1 = em[:, :d]
        e2 = em[:, d:]
        e64_ref[...] = e1
        w1t = w1t_ref[...]
        w2t = w2t_ref[...]
        b1 = b1_ref[...]
        b2 = b2_ref[...]
        y = jnp.maximum(jnp.dot(e1, w1t, preferred_element_type=jnp.float32) + b1, 0.0)
        dec_ref[...] = jnp.dot(y, w2t, preferred_element_type=jnp.float32) + b2
        ya = jnp.maximum(jnp.dot(e2, w1t, preferred_element_type=jnp.float32) + b1, 0.0)
        deca_ref[...] = jnp.dot(ya, w2t, preferred_element_type=jnp.float32) + b2

    return pl.pallas_call(
        body,
        grid=(n // bn,),
        in_specs=[pl.BlockSpec((NC, bn, d2), lambda i: (0, i, 0)),
                  pl.BlockSpec((d, d), lambda i: (0, 0)),
                  pl.BlockSpec((1, d), lambda i: (0, 0)),
                  pl.BlockSpec((d, d), lambda i: (0, 0)),
                  pl.BlockSpec((1, d), lambda i: (0, 0))],
        out_specs=[pl.BlockSpec((bn, d), lambda i: (i, 0)),
                   pl.BlockSpec((bn, d), lambda i: (i, 0)),
                   pl.BlockSpec((bn, d2), lambda i: (i, 0)),
                   pl.BlockSpec((bn, d), lambda i: (i, 0)),
                   pl.BlockSpec((bn, d), lambda i: (i, 0))],
        out_shape=[jax.ShapeDtypeStruct((n, d), jnp.float32),
                   jax.ShapeDtypeStruct((n, d), jnp.float32),
                   jax.ShapeDtypeStruct((n, d2), jnp.float32),
                   jax.ShapeDtypeStruct((n, d), jnp.float32),
                   jax.ShapeDtypeStruct((n, d), jnp.float32)],
        interpret=interpret,
    )(pa, dw1t, db1, dw2t, db2)


def _tc_k3(n, pb, pc, w2, *, interpret=False):
    d = pb.shape[2]
    d2 = pc.shape[2]
    din = w2.shape[1]
    bn = _block_rows(n)

    def body(pb_ref, pc_ref, w2_ref, h_ref, ret_ref, reta_ref):
        sagg = pb_ref[0] + pb_ref[1]
        h_ref[...] = jnp.dot(sagg, w2_ref[...], preferred_element_type=jnp.float32)
        g = pc_ref[0] + pc_ref[1]

        def norm_sig(x):
            nn = jnp.sqrt(jnp.sum(x * x, axis=1, keepdims=True))
            return jax.nn.sigmoid(x / jnp.maximum(nn, 1e-12))

        ret_ref[...] = norm_sig(g[:, :d])
        reta_ref[...] = norm_sig(g[:, d:])

    return pl.pallas_call(
        body,
        grid=(n // bn,),
        in_specs=[pl.BlockSpec((NC, bn, d), lambda i: (0, i, 0)),
                  pl.BlockSpec((NC, bn, d2), lambda i: (0, i, 0)),
                  pl.BlockSpec((d, din), lambda i: (0, 0))],
        out_specs=[pl.BlockSpec((bn, din), lambda i: (i, 0)),
                   pl.BlockSpec((bn, d), lambda i: (i, 0)),
                   pl.BlockSpec((bn, d), lambda i: (i, 0))],
        out_shape=[jax.ShapeDtypeStruct((n, din), jnp.float32),
                   jax.ShapeDtypeStruct((n, d), jnp.float32),
                   jax.ShapeDtypeStruct((n, d), jnp.float32)],
        interpret=interpret,
    )(pb, pc, w2)


def kernel(feat, feat_a, adj_indices, adj_values, neigh_indices, neigh_values,
           weight1, weight2, dec_w1, dec_b1, dec_w2, dec_b2):
    n = feat.shape[0]
    adj_i = adj_indices.astype(jnp.int32)
    nei_i = neigh_indices.astype(jnp.int32)
    pack_a80 = _pack_edges(adj_i[0], adj_i[1], 80)
    pack_a128 = _pack_edges(adj_i[0], adj_i[1], 128)
    pack_n80 = _pack_edges(nei_i[0], nei_i[1], 80)
    vals_a80 = adj_values.astype(jnp.float32).reshape(-1, 80)
    vals_a128 = adj_values.astype(jnp.float32).reshape(-1, 128)

    zc = _tc_k1(feat, feat_a, weight1)
    pa = _sc_spmm(zc, pack_a80, vals_a80, scaled=True, nbuf=3)
    hiden_emb, emb64, emb128, dec, dec_a = _tc_k2(
        n, pa, dec_w1.T, dec_b1.reshape(1, -1), dec_w2.T, dec_b2.reshape(1, -1))
    pb = _sc_spmm(emb64, pack_a128, vals_a128, scaled=True, nbuf=5)
    pc = _sc_spmm(emb128, pack_n80, None, scaled=False, nbuf=3)
    h, ret, ret_a = _tc_k3(n, pb, pc, weight2)
    return (hiden_emb, h, dec, dec_a, ret, ret_a)


# speedup vs baseline: 1.5735x; 1.0153x over previous
import jax
import jax.numpy as jnp
from jax import lax
from jax.experimental import pallas as pl
from jax.experimental.pallas import tpu as pltpu
from jax.experimental.pallas import tpu_sc as plsc

NC = 2
NS = 16
CHUNK = 128
NBUF = 4


def _sc_spmm(x, pack, vals, *, scaled, nbuf, interpret=False):
    n, d = x.shape
    npk, chunk = pack.shape[1], pack.shape[2]
    nchunks = pack.shape[0] - PAD_CHUNKS
    assert d % 16 == 0 and npk == 2 and (vals is not None) == scaled
    assert chunk % 16 == 0 and chunk <= 128 and nbuf <= PAD_CHUNKS
    n_pad = -(-n // (NS * 8)) * NS * 8
    rps = n_pad // NS
    q, r = divmod(nchunks, NC * NS)
    mesh = plsc.VectorSubcoreMesh(core_axis_name="c", subcore_axis_name="s",
                                  num_cores=NC, num_subcores=NS)

    def body(x_hbm, pack_hbm, *rest):
        it = iter(rest)
        vals_hbm = next(it) if scaled else None
        zeros_hbm = next(it)
        out_hbm = next(it)
        acc = next(it)
        idxs = [next(it) for _ in range(2)]
        vals_v = [next(it) for _ in range(2)] if scaled else None
        bufs = [next(it) for _ in range(nbuf)]
        semi = [next(it) for _ in range(2)]
        semg = [next(it) for _ in range(nbuf)]
        sems = [next(it) for _ in range(nbuf)]
        assert next(it, None) is None

        c = lax.axis_index("c")
        s = lax.axis_index("s")
        w = c * NS + s
        r0 = s * rps
        pltpu.sync_copy(zeros_hbm.at[pl.ds(r0, rps)], acc.at[pl.ds(r0, rps)])
        plsc.subcore_barrier()

        base = q * w + jnp.minimum(w, r)
        count = q + (w < r).astype(jnp.int32)

        def fire_idx(p, g):
            sl = pl.ds(base + g * nbuf, nbuf)
            pltpu.async_copy(pack_hbm.at[sl], idxs[p], semi[p])
            if scaled:
                pltpu.async_copy(vals_hbm.at[sl], vals_v[p], semi[p])

        def drain_idx(p, g):
            sl = pl.ds(base + g * nbuf, nbuf)
            pltpu.make_async_copy(pack_hbm.at[sl], idxs[p], semi[p]).wait()
            if scaled:
                pltpu.make_async_copy(vals_hbm.at[sl], vals_v[p], semi[p]).wait()

        def fire_gather(p, b):
            pltpu.async_copy(x_hbm.at[idxs[p].at[b, 1]], bufs[b], semg[b])

        def drain_gather(p, b):
            pltpu.make_async_copy(x_hbm.at[idxs[p].at[b, 1]], bufs[b], semg[b]).wait()

        def fire_scatter(p, b):
            pltpu.async_copy(bufs[b], acc.at[idxs[p].at[b, 0]], sems[b], add=True)

        def drain_scatter(b):
            pltpu.make_async_copy(bufs[b], acc.at[idxs[0].at[b, 0]], sems[b]).wait()

        def scale(p, b):
            def scale_group(g, cc):
                v16 = vals_v[p][b, pl.ds(g * 16, 16)]
                for j in range(16):
                    v = v16[j]
                    for k in range(d // 16):
                        sl = pl.ds(k * 16, 16)
                        bufs[b][g * 16 + j, sl] = bufs[b][g * 16 + j, sl] * v
                return cc

            lax.fori_loop(0, chunk // 16, scale_group, 0)

        @pl.when(0 < count)
        def _():
            fire_idx(0, 0)

        def run_group(gg, p):
            ch0 = gg * nbuf
            @pl.when(ch0 < count)
            def _():
                drain_idx(p, gg)

            for b in range(nbuf):
                ch = ch0 + b

                @pl.when(ch < count)
                def _(b=b, ch=ch):
                    @pl.when(ch >= nbuf)
                    def _():
                        drain_scatter(b)

                    fire_gather(p, b)

            @pl.when(ch0 + nbuf < count)
            def _():
                fire_idx(1 - p, gg + 1)

            for b in range(nbuf):
                ch = ch0 + b

                @pl.when(ch < count)
                def _(b=b, ch=ch):
                    drain_gather(p, b)
                    if scaled:
                        scale(p, b)
                    fire_scatter(p, b)

        def outer(g, carry):
            run_group(2 * g, 0)
            run_group(2 * g + 1, 1)
            return carry

        lax.fori_loop(0, (count + 2 * nbuf - 1) // (2 * nbuf), outer, 0)

        for b in range(nbuf):
            @pl.when(b < count)
            def _(b=b):
                drain_scatter(b)

        plsc.subcore_barrier()
        pltpu.sync_copy(acc.at[pl.ds(r0, rps)], out_hbm.at[c, pl.ds(r0, rps)])

    out_type = jax.ShapeDtypeStruct((NC, n_pad, d), jnp.float32)
    scratch = [pltpu.VMEM_SHARED((n_pad, d), jnp.float32)]
    scratch += [pltpu.VMEM((nbuf, npk, chunk), jnp.int32) for _ in range(2)]
    if scaled:
        scratch += [pltpu.VMEM((nbuf, chunk), jnp.float32) for _ in range(2)]
    scratch += [pltpu.VMEM((chunk, d), jnp.float32) for _ in range(nbuf)]
    scratch += [pltpu.SemaphoreType.DMA for _ in range(2 + 2 * nbuf)]

    kfn = pl.kernel(
        body,
        out_type=out_type,
        mesh=mesh,
        scratch_types=scratch,
        compiler_params=pltpu.CompilerParams(use_tc_tiling_on_sc=False),
        interpret=interpret,
    )
    args = [x, pack]
    if scaled:
        args.append(vals)
    args.append(jnp.zeros((n_pad, d), jnp.float32))
    return kfn(*args)


PAD_CHUNKS = 8


def _pack_edges(rows, cols, chunk):
    p = jnp.concatenate(
        [rows.reshape(-1, 1, chunk), cols.reshape(-1, 1, chunk)], axis=1)
    return jnp.concatenate(
        [p, jnp.zeros((PAD_CHUNKS, 2, chunk), jnp.int32)], axis=0)


def _pad_vals(vals, chunk):
    v = vals.astype(jnp.float32).reshape(-1, chunk)
    return jnp.concatenate([v, jnp.zeros((PAD_CHUNKS, chunk), jnp.float32)], axis=0)


def _block_rows(n):
    for cand in (2000, 1000, 500, 200, 104, 80, 40, 16, 8):
        if n % cand == 0:
            return cand
    return n


def _tc_k1(feat, feat_a, w1, *, interpret=False):
    n, din = feat.shape
    dout = w1.shape[1]
    bn = _block_rows(n)

    def body(f_ref, fa_ref, w_ref, o_ref):
        w = w_ref[...]
        o_ref[:, :dout] = jnp.dot(f_ref[...], w, preferred_element_type=jnp.float32)
        o_ref[:, dout:] = jnp.dot(fa_ref[...], w, preferred_element_type=jnp.float32)

    return pl.pallas_call(
        body,
        grid=(n // bn,),
        in_specs=[pl.BlockSpec((bn, din), lambda i: (i, 0)),
                  pl.BlockSpec((bn, din), lambda i: (i, 0)),
                  pl.BlockSpec((din, dout), lambda i: (0, 0))],
        out_specs=pl.BlockSpec((bn, 2 * dout), lambda i: (i, 0)),
        out_shape=jax.ShapeDtypeStruct((n, 2 * dout), jnp.float32),
        interpret=interpret,
    )(feat, feat_a, w1)


def _tc_k2(n, pa, dw1t, db1, dw2t, db2, *, interpret=False):
    d2 = pa.shape[2]
    d = d2 // 2
    bn = _block_rows(n)

    def body(pa_ref, w1t_ref, b1_ref, w2t_ref, b2_ref,
             hid_ref, e64_ref, e128_ref, dec_ref, deca_ref):
        z = pa_ref[0] + pa_ref[1]
        hid_ref[...] = z[:, :d]
        em = jnp.maximum(z, 0.0)
        e128_ref[...] = em
        e1 = em[:, :d]
        e2 = em[:, d:]
        e64_ref[...] = e1
        w1t = w1t_ref[...]
        w2t = w2t_ref[...]
        b1 = b1_ref[...]
        b2 = b2_ref[...]
        y = jnp.maximum(jnp.dot(e1, w1t, preferred_element_type=jnp.float32) + b1, 0.0)
        dec_ref[...] = jnp.dot(y, w2t, preferred_element_type=jnp.float32) + b2
        ya = jnp.maximum(jnp.dot(e2, w1t, preferred_element_type=jnp.float32) + b1, 0.0)
        deca_ref[...] = jnp.dot(ya, w2t, preferred_element_type=jnp.float32) + b2

    return pl.pallas_call(
        body,
        grid=(n // bn,),
        in_specs=[pl.BlockSpec((NC, bn, d2), lambda i: (0, i, 0)),
                  pl.BlockSpec((d, d), lambda i: (0, 0)),
                  pl.BlockSpec((1, d), lambda i: (0, 0)),
                  pl.BlockSpec((d, d), lambda i: (0, 0)),
                  pl.BlockSpec((1, d), lambda i: (0, 0))],
        out_specs=[pl.BlockSpec((bn, d), lambda i: (i, 0)),
                   pl.BlockSpec((bn, d), lambda i: (i, 0)),
                   pl.BlockSpec((bn, d2), lambda i: (i, 0)),
                   pl.BlockSpec((bn, d), lambda i: (i, 0)),
                   pl.BlockSpec((bn, d), lambda i: (i, 0))],
        out_shape=[jax.ShapeDtypeStruct((n, d), jnp.float32),
                   jax.ShapeDtypeStruct((n, d), jnp.float32),
                   jax.ShapeDtypeStruct((n, d2), jnp.float32),
                   jax.ShapeDtypeStruct((n, d), jnp.float32),
                   jax.ShapeDtypeStruct((n, d), jnp.float32)],
        interpret=interpret,
    )(pa, dw1t, db1, dw2t, db2)


def _tc_k3(n, pb, pc, w2, *, interpret=False):
    d = pb.shape[2]
    d2 = pc.shape[2]
    din = w2.shape[1]
    bn = _block_rows(n)

    def body(pb_ref, pc_ref, w2_ref, h_ref, ret_ref, reta_ref):
        sagg = pb_ref[0] + pb_ref[1]
        h_ref[...] = jnp.dot(sagg, w2_ref[...], preferred_element_type=jnp.float32)
        g = pc_ref[0] + pc_ref[1]

        def norm_sig(x):
            nn = jnp.sqrt(jnp.sum(x * x, axis=1, keepdims=True))
            return jax.nn.sigmoid(x / jnp.maximum(nn, 1e-12))

        ret_ref[...] = norm_sig(g[:, :d])
        reta_ref[...] = norm_sig(g[:, d:])

    return pl.pallas_call(
        body,
        grid=(n // bn,),
        in_specs=[pl.BlockSpec((NC, bn, d), lambda i: (0, i, 0)),
                  pl.BlockSpec((NC, bn, d2), lambda i: (0, i, 0)),
                  pl.BlockSpec((d, din), lambda i: (0, 0))],
        out_specs=[pl.BlockSpec((bn, din), lambda i: (i, 0)),
                   pl.BlockSpec((bn, d), lambda i: (i, 0)),
                   pl.BlockSpec((bn, d), lambda i: (i, 0))],
        out_shape=[jax.ShapeDtypeStruct((n, din), jnp.float32),
                   jax.ShapeDtypeStruct((n, d), jnp.float32),
                   jax.ShapeDtypeStruct((n, d), jnp.float32)],
        interpret=interpret,
    )(pb, pc, w2)


def kernel(feat, feat_a, adj_indices, adj_values, neigh_indices, neigh_values,
           weight1, weight2, dec_w1, dec_b1, dec_w2, dec_b2):
    n = feat.shape[0]
    adj_i = adj_indices.astype(jnp.int32)
    nei_i = neigh_indices.astype(jnp.int32)

    pack_a128 = _pack_edges(adj_i[0], adj_i[1], 128)
    pack_n128 = _pack_edges(nei_i[0], nei_i[1], 128)
    vals_a128 = _pad_vals(adj_values, 128)

    zc = _tc_k1(feat, feat_a, weight1)
    pa = _sc_spmm(zc, pack_a128, vals_a128, scaled=True, nbuf=2)
    hiden_emb, emb64, emb128, dec, dec_a = _tc_k2(
        n, pa, dec_w1.T, dec_b1.reshape(1, -1), dec_w2.T, dec_b2.reshape(1, -1))
    pb = _sc_spmm(emb64, pack_a128, vals_a128, scaled=True, nbuf=5)
    pc = _sc_spmm(emb128, pack_n128, None, scaled=False, nbuf=2)
    h, ret, ret_a = _tc_k3(n, pb, pc, weight2)
    return (hiden_emb, h, dec, dec_a, ret, ret_a)


# speedup vs baseline: 1.5960x; 1.0143x over previous
import jax
import jax.numpy as jnp
from jax import lax
from jax.experimental import pallas as pl
from jax.experimental.pallas import tpu as pltpu
from jax.experimental.pallas import tpu_sc as plsc

NC = 2
NS = 16
CHUNK = 128
NBUF = 4


def _sc_spmm(x, pack, vals, *, scaled, nbuf, interpret=False):
    n, d = x.shape
    npk, chunk = pack.shape[1], pack.shape[2]
    nchunks = pack.shape[0] - PAD_CHUNKS
    assert d % 16 == 0 and npk == 2 and (vals is not None) == scaled
    assert chunk % 16 == 0 and chunk <= 128 and nbuf <= PAD_CHUNKS
    n_pad = -(-n // (NS * 8)) * NS * 8
    rps = n_pad // NS
    q, r = divmod(nchunks, NC * NS)
    mesh = plsc.VectorSubcoreMesh(core_axis_name="c", subcore_axis_name="s",
                                  num_cores=NC, num_subcores=NS)

    def body(x_hbm, pack_hbm, *rest):
        it = iter(rest)
        vals_hbm = next(it) if scaled else None
        zeros_hbm = next(it)
        out_hbm = next(it)
        acc = next(it)
        idxs = [next(it) for _ in range(2)]
        vals_v = [next(it) for _ in range(2)] if scaled else None
        bufs = [next(it) for _ in range(nbuf)]
        semi = [next(it) for _ in range(2)]
        semg = [next(it) for _ in range(nbuf)]
        sems = [next(it) for _ in range(nbuf)]
        assert next(it, None) is None

        c = lax.axis_index("c")
        s = lax.axis_index("s")
        w = c * NS + s
        r0 = s * rps
        pltpu.sync_copy(zeros_hbm.at[pl.ds(r0, rps)], acc.at[pl.ds(r0, rps)])
        plsc.subcore_barrier()

        base = q * w + jnp.minimum(w, r)
        count = q + (w < r).astype(jnp.int32)

        def fire_idx(p, g):
            sl = pl.ds(base + g * nbuf, nbuf)
            pltpu.async_copy(pack_hbm.at[sl], idxs[p], semi[p])
            if scaled:
                pltpu.async_copy(vals_hbm.at[sl], vals_v[p], semi[p])

        def drain_idx(p, g):
            sl = pl.ds(base + g * nbuf, nbuf)
            pltpu.make_async_copy(pack_hbm.at[sl], idxs[p], semi[p]).wait()
            if scaled:
                pltpu.make_async_copy(vals_hbm.at[sl], vals_v[p], semi[p]).wait()

        def fire_gather(p, b):
            pltpu.async_copy(x_hbm.at[idxs[p].at[b, 1]], bufs[b], semg[b])

        def drain_gather(p, b):
            pltpu.make_async_copy(x_hbm.at[idxs[p].at[b, 1]], bufs[b], semg[b]).wait()

        def fire_scatter(p, b):
            pltpu.async_copy(bufs[b], acc.at[idxs[p].at[b, 0]], sems[b], add=True)

        def drain_scatter(b):
            pltpu.make_async_copy(bufs[b], acc.at[idxs[0].at[b, 0]], sems[b]).wait()

        def scale(p, b):
            def scale_group(g, cc):
                v16 = vals_v[p][b, pl.ds(g * 16, 16)]
                for j in range(16):
                    v = v16[j]
                    for k in range(d // 16):
                        sl = pl.ds(k * 16, 16)
                        bufs[b][g * 16 + j, sl] = bufs[b][g * 16 + j, sl] * v
                return cc

            plsc.parallel_loop(0, chunk // 16)(lambda g: scale_group(g, None) and None)

        @pl.when(0 < count)
        def _():
            fire_idx(0, 0)

        def run_group(gg, p):
            ch0 = gg * nbuf
            @pl.when(ch0 < count)
            def _():
                drain_idx(p, gg)

            for b in range(nbuf):
                ch = ch0 + b

                @pl.when(ch < count)
                def _(b=b, ch=ch):
                    @pl.when(ch >= nbuf)
                    def _():
                        drain_scatter(b)

                    fire_gather(p, b)

            @pl.when(ch0 + nbuf < count)
            def _():
                fire_idx(1 - p, gg + 1)

            for b in range(nbuf):
                ch = ch0 + b

                @pl.when(ch < count)
                def _(b=b, ch=ch):
                    drain_gather(p, b)
                    if scaled:
                        scale(p, b)
                    fire_scatter(p, b)

        def outer(g, carry):
            run_group(2 * g, 0)
            run_group(2 * g + 1, 1)
            return carry

        lax.fori_loop(0, (count + 2 * nbuf - 1) // (2 * nbuf), outer, 0)

        for b in range(nbuf):
            @pl.when(b < count)
            def _(b=b):
                drain_scatter(b)

        plsc.subcore_barrier()
        pltpu.sync_copy(acc.at[pl.ds(r0, rps)], out_hbm.at[c, pl.ds(r0, rps)])

    out_type = jax.ShapeDtypeStruct((NC, n_pad, d), jnp.float32)
    scratch = [pltpu.VMEM_SHARED((n_pad, d), jnp.float32)]
    scratch += [pltpu.VMEM((nbuf, npk, chunk), jnp.int32) for _ in range(2)]
    if scaled:
        scratch += [pltpu.VMEM((nbuf, chunk), jnp.float32) for _ in range(2)]
    scratch += [pltpu.VMEM((chunk, d), jnp.float32) for _ in range(nbuf)]
    scratch += [pltpu.SemaphoreType.DMA for _ in range(2 + 2 * nbuf)]

    kfn = pl.kernel(
        body,
        out_type=out_type,
        mesh=mesh,
        scratch_types=scratch,
        compiler_params=pltpu.CompilerParams(use_tc_tiling_on_sc=False),
        interpret=interpret,
    )
    args = [x, pack]
    if scaled:
        args.append(vals)
    args.append(jnp.zeros((n_pad, d), jnp.float32))
    return kfn(*args)


PAD_CHUNKS = 8


def _pack_edges(rows, cols, chunk):
    p = jnp.concatenate(
        [rows.reshape(-1, 1, chunk), cols.reshape(-1, 1, chunk)], axis=1)
    return jnp.concatenate(
        [p, jnp.zeros((PAD_CHUNKS, 2, chunk), jnp.int32)], axis=0)


def _pad_vals(vals, chunk):
    v = vals.astype(jnp.float32).reshape(-1, chunk)
    return jnp.concatenate([v, jnp.zeros((PAD_CHUNKS, chunk), jnp.float32)], axis=0)


def _block_rows(n):
    for cand in (2000, 1000, 500, 200, 104, 80, 40, 16, 8):
        if n % cand == 0:
            return cand
    return n


def _tc_k1(feat, feat_a, w1, *, interpret=False):
    n, din = feat.shape
    dout = w1.shape[1]
    bn = _block_rows(n)

    def body(f_ref, fa_ref, w_ref, o_ref):
        w = w_ref[...]
        o_ref[:, :dout] = jnp.dot(f_ref[...], w, preferred_element_type=jnp.float32)
        o_ref[:, dout:] = jnp.dot(fa_ref[...], w, preferred_element_type=jnp.float32)

    return pl.pallas_call(
        body,
        grid=(n // bn,),
        in_specs=[pl.BlockSpec((bn, din), lambda i: (i, 0)),
                  pl.BlockSpec((bn, din), lambda i: (i, 0)),
                  pl.BlockSpec((din, dout), lambda i: (0, 0))],
        out_specs=pl.BlockSpec((bn, 2 * dout), lambda i: (i, 0)),
        out_shape=jax.ShapeDtypeStruct((n, 2 * dout), jnp.float32),
        interpret=interpret,
    )(feat, feat_a, w1)


def _tc_k2(n, pa, dw1t, db1, dw2t, db2, *, interpret=False):
    d2 = pa.shape[2]
    d = d2 // 2
    bn = _block_rows(n)

    def body(pa_ref, w1t_ref, b1_ref, w2t_ref, b2_ref,
             hid_ref, e64_ref, e128_ref, dec_ref, deca_ref):
        z = pa_ref[0] + pa_ref[1]
        hid_ref[...] = z[:, :d]
        em = jnp.maximum(z, 0.0)
        e128_ref[...] = em
        e1 = em[:, :d]
        e2 = em[:, d:]
        e64_ref[...] = e1
        w1t = w1t_ref[...]
        w2t = w2t_ref[...]
        b1 = b1_ref[...]
        b2 = b2_ref[...]
        y = jnp.maximum(jnp.dot(e1, w1t, preferred_element_type=jnp.float32) + b1, 0.0)
        dec_ref[...] = jnp.dot(y, w2t, preferred_element_type=jnp.float32) + b2
        ya = jnp.maximum(jnp.dot(e2, w1t, preferred_element_type=jnp.float32) + b1, 0.0)
        deca_ref[...] = jnp.dot(ya, w2t, preferred_element_type=jnp.float32) + b2

    return pl.pallas_call(
        body,
        grid=(n // bn,),
        in_specs=[pl.BlockSpec((NC, bn, d2), lambda i: (0, i, 0)),
                  pl.BlockSpec((d, d), lambda i: (0, 0)),
                  pl.BlockSpec((1, d), lambda i: (0, 0)),
                  pl.BlockSpec((d, d), lambda i: (0, 0)),
                  pl.BlockSpec((1, d), lambda i: (0, 0))],
        out_specs=[pl.BlockSpec((bn, d), lambda i: (i, 0)),
                   pl.BlockSpec((bn, d), lambda i: (i, 0)),
                   pl.BlockSpec((bn, d2), lambda i: (i, 0)),
                   pl.BlockSpec((bn, d), lambda i: (i, 0)),
                   pl.BlockSpec((bn, d), lambda i: (i, 0))],
        out_shape=[jax.ShapeDtypeStruct((n, d), jnp.float32),
                   jax.ShapeDtypeStruct((n, d), jnp.float32),
                   jax.ShapeDtypeStruct((n, d2), jnp.float32),
                   jax.ShapeDtypeStruct((n, d), jnp.float32),
                   jax.ShapeDtypeStruct((n, d), jnp.float32)],
        interpret=interpret,
    )(pa, dw1t, db1, dw2t, db2)


def _tc_k3(n, pb, pc, w2, *, interpret=False):
    d = pb.shape[2]
    d2 = pc.shape[2]
    din = w2.shape[1]
    bn = _block_rows(n)

    def body(pb_ref, pc_ref, w2_ref, h_ref, ret_ref, reta_ref):
        sagg = pb_ref[0] + pb_ref[1]
        h_ref[...] = jnp.dot(sagg, w2_ref[...], preferred_element_type=jnp.float32)
        g = pc_ref[0] + pc_ref[1]

        def norm_sig(x):
            nn = jnp.sqrt(jnp.sum(x * x, axis=1, keepdims=True))
            return jax.nn.sigmoid(x / jnp.maximum(nn, 1e-12))

        ret_ref[...] = norm_sig(g[:, :d])
        reta_ref[...] = norm_sig(g[:, d:])

    return pl.pallas_call(
        body,
        grid=(n // bn,),
        in_specs=[pl.BlockSpec((NC, bn, d), lambda i: (0, i, 0)),
                  pl.BlockSpec((NC, bn, d2), lambda i: (0, i, 0)),
                  pl.BlockSpec((d, din), lambda i: (0, 0))],
        out_specs=[pl.BlockSpec((bn, din), lambda i: (i, 0)),
                   pl.BlockSpec((bn, d), lambda i: (i, 0)),
                   pl.BlockSpec((bn, d), lambda i: (i, 0))],
        out_shape=[jax.ShapeDtypeStruct((n, din), jnp.float32),
                   jax.ShapeDtypeStruct((n, d), jnp.float32),
                   jax.ShapeDtypeStruct((n, d), jnp.float32)],
        interpret=interpret,
    )(pb, pc, w2)


def kernel(feat, feat_a, adj_indices, adj_values, neigh_indices, neigh_values,
           weight1, weight2, dec_w1, dec_b1, dec_w2, dec_b2):
    n = feat.shape[0]
    adj_i = adj_indices.astype(jnp.int32)
    nei_i = neigh_indices.astype(jnp.int32)

    pack_a128 = _pack_edges(adj_i[0], adj_i[1], 128)
    pack_n128 = _pack_edges(nei_i[0], nei_i[1], 128)
    vals_a128 = _pad_vals(adj_values, 128)

    zc = _tc_k1(feat, feat_a, weight1)
    pa = _sc_spmm(zc, pack_a128, vals_a128, scaled=True, nbuf=2)
    hiden_emb, emb64, emb128, dec, dec_a = _tc_k2(
        n, pa, dec_w1.T, dec_b1.reshape(1, -1), dec_w2.T, dec_b2.reshape(1, -1))
    pb = _sc_spmm(emb64, pack_a128, vals_a128, scaled=True, nbuf=5)
    pc = _sc_spmm(emb128, pack_n128, None, scaled=False, nbuf=2)
    h, ret, ret_a = _tc_k3(n, pb, pc, weight2)
    return (hiden_emb, h, dec, dec_a, ret, ret_a)


# speedup vs baseline: 1.6386x; 1.0267x over previous
import jax
import jax.numpy as jnp
from jax import lax
from jax.experimental import pallas as pl
from jax.experimental.pallas import tpu as pltpu
from jax.experimental.pallas import tpu_sc as plsc

NC = 2
NS = 16
CHUNK = 128
NBUF = 4


def _sc_spmm(x, pack, vals, *, scaled, nbuf, interpret=False):
    n, d = x.shape
    npk, chunk = pack.shape[1], pack.shape[2]
    nchunks = pack.shape[0] - PAD_CHUNKS
    assert d % 16 == 0 and npk == 2 and (vals is not None) == scaled
    assert chunk % 16 == 0 and chunk <= 128 and nbuf <= PAD_CHUNKS
    n_pad = -(-n // (NS * 8)) * NS * 8
    rps = n_pad // NS
    q, r = divmod(nchunks, NC * NS)
    mesh = plsc.VectorSubcoreMesh(core_axis_name="c", subcore_axis_name="s",
                                  num_cores=NC, num_subcores=NS)

    def body(x_hbm, pack_hbm, *rest):
        it = iter(rest)
        vals_hbm = next(it) if scaled else None
        out_hbm = next(it)
        acc = next(it)
        idxs = [next(it) for _ in range(2)]
        vals_v = [next(it) for _ in range(2)] if scaled else None
        bufs = [next(it) for _ in range(nbuf)]
        semi = [next(it) for _ in range(2)]
        semg = [next(it) for _ in range(nbuf)]
        sems = [next(it) for _ in range(nbuf)]
        assert next(it, None) is None

        c = lax.axis_index("c")
        s = lax.axis_index("s")
        w = c * NS + s
        r0 = s * rps
        zero16 = jnp.zeros((16,), jnp.float32)

        @plsc.parallel_loop(0, chunk)
        def _(rr):
            for k in range(d // 16):
                bufs[0][rr, pl.ds(k * 16, 16)] = zero16

        nfull, tail = divmod(rps, chunk)
        for t in range(nfull):
            pltpu.sync_copy(bufs[0].at[pl.ds(0, chunk)],
                            acc.at[pl.ds(r0 + t * chunk, chunk)])
        if tail:
            pltpu.sync_copy(bufs[0].at[pl.ds(0, tail)],
                            acc.at[pl.ds(r0 + nfull * chunk, tail)])
        plsc.subcore_barrier()

        base = q * w + jnp.minimum(w, r)
        count = q + (w < r).astype(jnp.int32)

        def fire_idx(p, g):
            sl = pl.ds(base + g * nbuf, nbuf)
            pltpu.async_copy(pack_hbm.at[sl], idxs[p], semi[p])
            if scaled:
                pltpu.async_copy(vals_hbm.at[sl], vals_v[p], semi[p])

        def drain_idx(p, g):
            sl = pl.ds(base + g * nbuf, nbuf)
            pltpu.make_async_copy(pack_hbm.at[sl], idxs[p], semi[p]).wait()
            if scaled:
                pltpu.make_async_copy(vals_hbm.at[sl], vals_v[p], semi[p]).wait()

        def fire_gather(p, b):
            pltpu.async_copy(x_hbm.at[idxs[p].at[b, 1]], bufs[b], semg[b])

        def drain_gather(p, b):
            pltpu.make_async_copy(x_hbm.at[idxs[p].at[b, 1]], bufs[b], semg[b]).wait()

        def fire_scatter(p, b):
            pltpu.async_copy(bufs[b], acc.at[idxs[p].at[b, 0]], sems[b], add=True)

        def drain_scatter(b):
            pltpu.make_async_copy(bufs[b], acc.at[idxs[0].at[b, 0]], sems[b]).wait()

        def scale(p, b):
            def scale_group(g, cc):
                v16 = vals_v[p][b, pl.ds(g * 16, 16)]
                for j in range(16):
                    v = v16[j]
                    for k in range(d // 16):
                        sl = pl.ds(k * 16, 16)
                        bufs[b][g * 16 + j, sl] = bufs[b][g * 16 + j, sl] * v
                return cc

            plsc.parallel_loop(0, chunk // 16)(lambda g: scale_group(g, None) and None)

        @pl.when(0 < count)
        def _():
            fire_idx(0, 0)

        def run_group(gg, p):
            ch0 = gg * nbuf
            @pl.when(ch0 < count)
            def _():
                drain_idx(p, gg)

            for b in range(nbuf):
                ch = ch0 + b

                @pl.when(ch < count)
                def _(b=b, ch=ch):
                    @pl.when(ch >= nbuf)
                    def _():
                        drain_scatter(b)

                    fire_gather(p, b)

            @pl.when(ch0 + nbuf < count)
            def _():
                fire_idx(1 - p, gg + 1)

            for b in range(nbuf):
                ch = ch0 + b

                @pl.when(ch < count)
                def _(b=b, ch=ch):
                    drain_gather(p, b)
                    if scaled:
                        scale(p, b)
                    fire_scatter(p, b)

        def outer(g, carry):
            run_group(2 * g, 0)
            run_group(2 * g + 1, 1)
            return carry

        lax.fori_loop(0, (count + 2 * nbuf - 1) // (2 * nbuf), outer, 0)

        for b in range(nbuf):
            @pl.when(b < count)
            def _(b=b):
                drain_scatter(b)

        plsc.subcore_barrier()
        pltpu.sync_copy(acc.at[pl.ds(r0, rps)], out_hbm.at[c, pl.ds(r0, rps)])

    out_type = jax.ShapeDtypeStruct((NC, n_pad, d), jnp.float32)
    scratch = [pltpu.VMEM_SHARED((n_pad, d), jnp.float32)]
    scratch += [pltpu.VMEM((nbuf, npk, chunk), jnp.int32) for _ in range(2)]
    if scaled:
        scratch += [pltpu.VMEM((nbuf, chunk), jnp.float32) for _ in range(2)]
    scratch += [pltpu.VMEM((chunk, d), jnp.float32) for _ in range(nbuf)]
    scratch += [pltpu.SemaphoreType.DMA for _ in range(2 + 2 * nbuf)]

    kfn = pl.kernel(
        body,
        out_type=out_type,
        mesh=mesh,
        scratch_types=scratch,
        compiler_params=pltpu.CompilerParams(use_tc_tiling_on_sc=False),
        interpret=interpret,
    )
    args = [x, pack]
    if scaled:
        args.append(vals)
    return kfn(*args)


PAD_CHUNKS = 8


def _pack_edges(rows, cols, chunk):
    p = jnp.concatenate(
        [rows.reshape(-1, 1, chunk), cols.reshape(-1, 1, chunk)], axis=1)
    return jnp.concatenate(
        [p, jnp.zeros((PAD_CHUNKS, 2, chunk), jnp.int32)], axis=0)


def _pad_vals(vals, chunk):
    v = vals.astype(jnp.float32).reshape(-1, chunk)
    return jnp.concatenate([v, jnp.zeros((PAD_CHUNKS, chunk), jnp.float32)], axis=0)


def _block_rows(n):
    for cand in (2000, 1000, 500, 200, 104, 80, 40, 16, 8):
        if n % cand == 0:
            return cand
    return n


def _tc_k1(feat, feat_a, w1, *, interpret=False):
    n, din = feat.shape
    dout = w1.shape[1]
    bn = _block_rows(n)

    def body(f_ref, fa_ref, w_ref, o_ref):
        w = w_ref[...]
        o_ref[:, :dout] = jnp.dot(f_ref[...], w, preferred_element_type=jnp.float32)
        o_ref[:, dout:] = jnp.dot(fa_ref[...], w, preferred_element_type=jnp.float32)

    return pl.pallas_call(
        body,
        grid=(n // bn,),
        in_specs=[pl.BlockSpec((bn, din), lambda i: (i, 0)),
                  pl.BlockSpec((bn, din), lambda i: (i, 0)),
                  pl.BlockSpec((din, dout), lambda i: (0, 0))],
        out_specs=pl.BlockSpec((bn, 2 * dout), lambda i: (i, 0)),
        out_shape=jax.ShapeDtypeStruct((n, 2 * dout), jnp.float32),
        interpret=interpret,
    )(feat, feat_a, w1)


def _tc_k2(n, pa, dw1t, db1, dw2t, db2, *, interpret=False):
    d2 = pa.shape[2]
    d = d2 // 2
    bn = _block_rows(n)

    def body(pa_ref, w1t_ref, b1_ref, w2t_ref, b2_ref,
             hid_ref, e64_ref, e128_ref, dec_ref, deca_ref):
        z = pa_ref[0] + pa_ref[1]
        hid_ref[...] = z[:, :d]
        em = jnp.maximum(z, 0.0)
        e128_ref[...] = em
        e1 = em[:, :d]
        e2 = em[:, d:]
        e64_ref[...] = e1
        w1t = w1t_ref[...]
        w2t = w2t_ref[...]
        b1 = b1_ref[...]
        b2 = b2_ref[...]
        y = jnp.maximum(jnp.dot(e1, w1t, preferred_element_type=jnp.float32) + b1, 0.0)
        dec_ref[...] = jnp.dot(y, w2t, preferred_element_type=jnp.float32) + b2
        ya = jnp.maximum(jnp.dot(e2, w1t, preferred_element_type=jnp.float32) + b1, 0.0)
        deca_ref[...] = jnp.dot(ya, w2t, preferred_element_type=jnp.float32) + b2

    return pl.pallas_call(
        body,
        grid=(n // bn,),
        in_specs=[pl.BlockSpec((NC, bn, d2), lambda i: (0, i, 0)),
                  pl.BlockSpec((d, d), lambda i: (0, 0)),
                  pl.BlockSpec((1, d), lambda i: (0, 0)),
                  pl.BlockSpec((d, d), lambda i: (0, 0)),
                  pl.BlockSpec((1, d), lambda i: (0, 0))],
        out_specs=[pl.BlockSpec((bn, d), lambda i: (i, 0)),
                   pl.BlockSpec((bn, d), lambda i: (i, 0)),
                   pl.BlockSpec((bn, d2), lambda i: (i, 0)),
                   pl.BlockSpec((bn, d), lambda i: (i, 0)),
                   pl.BlockSpec((bn, d), lambda i: (i, 0))],
        out_shape=[jax.ShapeDtypeStruct((n, d), jnp.float32),
                   jax.ShapeDtypeStruct((n, d), jnp.float32),
                   jax.ShapeDtypeStruct((n, d2), jnp.float32),
                   jax.ShapeDtypeStruct((n, d), jnp.float32),
                   jax.ShapeDtypeStruct((n, d), jnp.float32)],
        interpret=interpret,
    )(pa, dw1t, db1, dw2t, db2)


def _tc_k3(n, pb, pc, w2, *, interpret=False):
    d = pb.shape[2]
    d2 = pc.shape[2]
    din = w2.shape[1]
    bn = _block_rows(n)

    def body(pb_ref, pc_ref, w2_ref, h_ref, ret_ref, reta_ref):
        sagg = pb_ref[0] + pb_ref[1]
        h_ref[...] = jnp.dot(sagg, w2_ref[...], preferred_element_type=jnp.float32)
        g = pc_ref[0] + pc_ref[1]

        def norm_sig(x):
            nn = jnp.sqrt(jnp.sum(x * x, axis=1, keepdims=True))
            return jax.nn.sigmoid(x / jnp.maximum(nn, 1e-12))

        ret_ref[...] = norm_sig(g[:, :d])
        reta_ref[...] = norm_sig(g[:, d:])

    return pl.pallas_call(
        body,
        grid=(n // bn,),
        in_specs=[pl.BlockSpec((NC, bn, d), lambda i: (0, i, 0)),
                  pl.BlockSpec((NC, bn, d2), lambda i: (0, i, 0)),
                  pl.BlockSpec((d, din), lambda i: (0, 0))],
        out_specs=[pl.BlockSpec((bn, din), lambda i: (i, 0)),
                   pl.BlockSpec((bn, d), lambda i: (i, 0)),
                   pl.BlockSpec((bn, d), lambda i: (i, 0))],
        out_shape=[jax.ShapeDtypeStruct((n, din), jnp.float32),
                   jax.ShapeDtypeStruct((n, d), jnp.float32),
                   jax.ShapeDtypeStruct((n, d), jnp.float32)],
        interpret=interpret,
    )(pb, pc, w2)


def kernel(feat, feat_a, adj_indices, adj_values, neigh_indices, neigh_values,
           weight1, weight2, dec_w1, dec_b1, dec_w2, dec_b2):
    n = feat.shape[0]
    adj_i = adj_indices.astype(jnp.int32)
    nei_i = neigh_indices.astype(jnp.int32)

    pack_a128 = _pack_edges(adj_i[0], adj_i[1], 128)
    pack_n128 = _pack_edges(nei_i[0], nei_i[1], 128)
    vals_a128 = _pad_vals(adj_values, 128)

    zc = _tc_k1(feat, feat_a, weight1)
    pa = _sc_spmm(zc, pack_a128, vals_a128, scaled=True, nbuf=2)
    hiden_emb, emb64, emb128, dec, dec_a = _tc_k2(
        n, pa, dec_w1.T, dec_b1.reshape(1, -1), dec_w2.T, dec_b2.reshape(1, -1))
    pb = _sc_spmm(emb64, pack_a128, vals_a128, scaled=True, nbuf=5)
    pc = _sc_spmm(emb128, pack_n128, None, scaled=False, nbuf=2)
    h, ret, ret_a = _tc_k3(n, pb, pc, weight2)
    return (hiden_emb, h, dec, dec_a, ret, ret_a)


# speedup vs baseline: 1.6394x; 1.0005x over previous
import jax
import jax.numpy as jnp
from jax import lax
from jax.experimental import pallas as pl
from jax.experimental.pallas import tpu as pltpu
from jax.experimental.pallas import tpu_sc as plsc

NC = 2
NS = 16


def _sc_spmm(x, pack, vals, *, scaled, nbuf, interpret=False):
    n, d = x.shape
    npk, chunk = pack.shape[1], pack.shape[2]
    nchunks = pack.shape[0] - PAD_CHUNKS
    assert d % 16 == 0 and npk == 2 and (vals is not None) == scaled
    assert chunk % 16 == 0 and chunk <= 128 and nbuf <= PAD_CHUNKS
    n_pad = -(-n // (NS * 8)) * NS * 8
    rps = n_pad // NS
    q, r = divmod(nchunks, NC * NS)
    mesh = plsc.VectorSubcoreMesh(core_axis_name="c", subcore_axis_name="s",
                                  num_cores=NC, num_subcores=NS)

    def body(x_hbm, pack_hbm, *rest):
        it = iter(rest)
        vals_hbm = next(it) if scaled else None
        out_hbm = next(it)
        acc = next(it)
        idxs = [next(it) for _ in range(2)]
        vals_v = [next(it) for _ in range(2)] if scaled else None
        bufs = [next(it) for _ in range(nbuf)]
        semi = [next(it) for _ in range(2)]
        semg = [next(it) for _ in range(nbuf)]
        sems = [next(it) for _ in range(nbuf)]
        assert next(it, None) is None

        c = lax.axis_index("c")
        s = lax.axis_index("s")
        w = c * NS + s
        r0 = s * rps
        zero16 = jnp.zeros((16,), jnp.float32)

        @plsc.parallel_loop(0, chunk)
        def _(rr):
            for k in range(d // 16):
                bufs[0][rr, pl.ds(k * 16, 16)] = zero16

        nfull, tail = divmod(rps, chunk)
        for t in range(nfull):
            pltpu.sync_copy(bufs[0].at[pl.ds(0, chunk)],
                            acc.at[pl.ds(r0 + t * chunk, chunk)])
        if tail:
            pltpu.sync_copy(bufs[0].at[pl.ds(0, tail)],
                            acc.at[pl.ds(r0 + nfull * chunk, tail)])
        plsc.subcore_barrier()

        base = q * w + jnp.minimum(w, r)
        count = q + (w < r).astype(jnp.int32)

        def fire_idx(p, g):
            sl = pl.ds(base + g * nbuf, nbuf)
            pltpu.async_copy(pack_hbm.at[sl], idxs[p], semi[p])
            if scaled:
                pltpu.async_copy(vals_hbm.at[sl], vals_v[p], semi[p])

        def drain_idx(p, g):
            sl = pl.ds(base + g * nbuf, nbuf)
            pltpu.make_async_copy(pack_hbm.at[sl], idxs[p], semi[p]).wait()
            if scaled:
                pltpu.make_async_copy(vals_hbm.at[sl], vals_v[p], semi[p]).wait()

        def fire_gather(p, b):
            pltpu.async_copy(x_hbm.at[idxs[p].at[b, 1]], bufs[b], semg[b])

        def drain_gather(p, b):
            pltpu.make_async_copy(x_hbm.at[idxs[p].at[b, 1]], bufs[b], semg[b]).wait()

        def fire_scatter(p, b):
            pltpu.async_copy(bufs[b], acc.at[idxs[p].at[b, 0]], sems[b], add=True)

        def drain_scatter(b):
            pltpu.make_async_copy(bufs[b], acc.at[idxs[0].at[b, 0]], sems[b]).wait()

        def scale(p, b):
            def scale_group(g, cc):
                v16 = vals_v[p][b, pl.ds(g * 16, 16)]
                for j in range(16):
                    v = v16[j]
                    for k in range(d // 16):
                        sl = pl.ds(k * 16, 16)
                        bufs[b][g * 16 + j, sl] = bufs[b][g * 16 + j, sl] * v
                return cc

            plsc.parallel_loop(0, chunk // 16)(lambda g: scale_group(g, None) and None)

        @pl.when(0 < count)
        def _():
            fire_idx(0, 0)

        def run_group(gg, p):
            ch0 = gg * nbuf
            @pl.when(ch0 < count)
            def _():
                drain_idx(p, gg)

            for b in range(nbuf):
                ch = ch0 + b

                @pl.when(ch < count)
                def _(b=b, ch=ch):
                    @pl.when(ch >= nbuf)
                    def _():
                        drain_scatter(b)

                    fire_gather(p, b)

            @pl.when(ch0 + nbuf < count)
            def _():
                fire_idx(1 - p, gg + 1)

            for b in range(nbuf):
                ch = ch0 + b

                @pl.when(ch < count)
                def _(b=b, ch=ch):
                    drain_gather(p, b)
                    if scaled:
                        scale(p, b)
                    fire_scatter(p, b)

        def outer(g, carry):
            run_group(2 * g, 0)
            run_group(2 * g + 1, 1)
            return carry

        lax.fori_loop(0, (count + 2 * nbuf - 1) // (2 * nbuf), outer, 0)

        for b in range(nbuf):
            @pl.when(b < count)
            def _(b=b):
                drain_scatter(b)

        plsc.subcore_barrier()
        pltpu.sync_copy(acc.at[pl.ds(r0, rps)], out_hbm.at[c, pl.ds(r0, rps)])

    out_type = jax.ShapeDtypeStruct((NC, n_pad, d), jnp.float32)
    scratch = [pltpu.VMEM_SHARED((n_pad, d), jnp.float32)]
    scratch += [pltpu.VMEM((nbuf, npk, chunk), jnp.int32) for _ in range(2)]
    if scaled:
        scratch += [pltpu.VMEM((nbuf, chunk), jnp.float32) for _ in range(2)]
    scratch += [pltpu.VMEM((chunk, d), jnp.float32) for _ in range(nbuf)]
    scratch += [pltpu.SemaphoreType.DMA for _ in range(2 + 2 * nbuf)]

    kfn = pl.kernel(
        body,
        out_type=out_type,
        mesh=mesh,
        scratch_types=scratch,
        compiler_params=pltpu.CompilerParams(use_tc_tiling_on_sc=False),
        interpret=interpret,
    )
    args = [x, pack]
    if scaled:
        args.append(vals)
    return kfn(*args)


PAD_CHUNKS = 8


def _pack_edges(rows, cols, chunk):
    p = jnp.concatenate(
        [rows.reshape(-1, 1, chunk), cols.reshape(-1, 1, chunk)], axis=1)
    return jnp.concatenate(
        [p, jnp.zeros((PAD_CHUNKS, 2, chunk), jnp.int32)], axis=0)


def _pad_vals(vals, chunk):
    v = vals.astype(jnp.float32).reshape(-1, chunk)
    return jnp.concatenate([v, jnp.zeros((PAD_CHUNKS, chunk), jnp.float32)], axis=0)


def _block_rows(n):
    for cand in (2000, 1000, 500, 200, 104, 80, 40, 16, 8):
        if n % cand == 0:
            return cand
    return n


def _tc_k1(feat, feat_a, w1, *, interpret=False):
    n, din = feat.shape
    dout = w1.shape[1]
    bn = _block_rows(n)

    def body(f_ref, fa_ref, w_ref, o_ref):
        w = w_ref[...]
        o_ref[:, :dout] = jnp.dot(f_ref[...], w, preferred_element_type=jnp.float32)
        o_ref[:, dout:] = jnp.dot(fa_ref[...], w, preferred_element_type=jnp.float32)

    return pl.pallas_call(
        body,
        grid=(n // bn,),
        in_specs=[pl.BlockSpec((bn, din), lambda i: (i, 0)),
                  pl.BlockSpec((bn, din), lambda i: (i, 0)),
                  pl.BlockSpec((din, dout), lambda i: (0, 0))],
        out_specs=pl.BlockSpec((bn, 2 * dout), lambda i: (i, 0)),
        out_shape=jax.ShapeDtypeStruct((n, 2 * dout), jnp.float32),
        interpret=interpret,
    )(feat, feat_a, w1)


def _tc_k2(n, pa, dw1t, db1, dw2t, db2, *, interpret=False):
    d2 = pa.shape[2]
    d = d2 // 2
    bn = _block_rows(n)

    def body(pa_ref, w1t_ref, b1_ref, w2t_ref, b2_ref,
             hid_ref, e64_ref, e128_ref, dec_ref, deca_ref):
        z = pa_ref[0] + pa_ref[1]
        hid_ref[...] = z[:, :d]
        em = jnp.maximum(z, 0.0)
        e128_ref[...] = em
        e1 = em[:, :d]
        e2 = em[:, d:]
        e64_ref[...] = e1
        w1t = w1t_ref[...]
        w2t = w2t_ref[...]
        b1 = b1_ref[...]
        b2 = b2_ref[...]
        y = jnp.maximum(jnp.dot(e1, w1t, preferred_element_type=jnp.float32) + b1, 0.0)
        dec_ref[...] = jnp.dot(y, w2t, preferred_element_type=jnp.float32) + b2
        ya = jnp.maximum(jnp.dot(e2, w1t, preferred_element_type=jnp.float32) + b1, 0.0)
        deca_ref[...] = jnp.dot(ya, w2t, preferred_element_type=jnp.float32) + b2

    return pl.pallas_call(
        body,
        grid=(n // bn,),
        in_specs=[pl.BlockSpec((NC, bn, d2), lambda i: (0, i, 0)),
                  pl.BlockSpec((d, d), lambda i: (0, 0)),
                  pl.BlockSpec((1, d), lambda i: (0, 0)),
                  pl.BlockSpec((d, d), lambda i: (0, 0)),
                  pl.BlockSpec((1, d), lambda i: (0, 0))],
        out_specs=[pl.BlockSpec((bn, d), lambda i: (i, 0)),
                   pl.BlockSpec((bn, d), lambda i: (i, 0)),
                   pl.BlockSpec((bn, d2), lambda i: (i, 0)),
                   pl.BlockSpec((bn, d), lambda i: (i, 0)),
                   pl.BlockSpec((bn, d), lambda i: (i, 0))],
        out_shape=[jax.ShapeDtypeStruct((n, d), jnp.float32),
                   jax.ShapeDtypeStruct((n, d), jnp.float32),
                   jax.ShapeDtypeStruct((n, d2), jnp.float32),
                   jax.ShapeDtypeStruct((n, d), jnp.float32),
                   jax.ShapeDtypeStruct((n, d), jnp.float32)],
        interpret=interpret,
    )(pa, dw1t, db1, dw2t, db2)


def _tc_k3(n, pb, pc, w2, *, interpret=False):
    d = pb.shape[2]
    d2 = pc.shape[2]
    din = w2.shape[1]
    bn = _block_rows(n)

    def body(pb_ref, pc_ref, w2_ref, h_ref, ret_ref, reta_ref):
        sagg = pb_ref[0] + pb_ref[1]
        h_ref[...] = jnp.dot(sagg, w2_ref[...], preferred_element_type=jnp.float32)
        g = pc_ref[0] + pc_ref[1]

        def norm_sig(x):
            nn = jnp.sqrt(jnp.sum(x * x, axis=1, keepdims=True))
            return jax.nn.sigmoid(x / jnp.maximum(nn, 1e-12))

        ret_ref[...] = norm_sig(g[:, :d])
        reta_ref[...] = norm_sig(g[:, d:])

    return pl.pallas_call(
        body,
        grid=(n // bn,),
        in_specs=[pl.BlockSpec((NC, bn, d), lambda i: (0, i, 0)),
                  pl.BlockSpec((NC, bn, d2), lambda i: (0, i, 0)),
                  pl.BlockSpec((d, din), lambda i: (0, 0))],
        out_specs=[pl.BlockSpec((bn, din), lambda i: (i, 0)),
                   pl.BlockSpec((bn, d), lambda i: (i, 0)),
                   pl.BlockSpec((bn, d), lambda i: (i, 0))],
        out_shape=[jax.ShapeDtypeStruct((n, din), jnp.float32),
                   jax.ShapeDtypeStruct((n, d), jnp.float32),
                   jax.ShapeDtypeStruct((n, d), jnp.float32)],
        interpret=interpret,
    )(pb, pc, w2)


def kernel(feat, feat_a, adj_indices, adj_values, neigh_indices, neigh_values,
           weight1, weight2, dec_w1, dec_b1, dec_w2, dec_b2):
    n = feat.shape[0]
    adj_i = adj_indices.astype(jnp.int32)
    nei_i = neigh_indices.astype(jnp.int32)

    pack_a128 = _pack_edges(adj_i[0], adj_i[1], 128)
    pack_n128 = _pack_edges(nei_i[0], nei_i[1], 128)
    vals_a128 = _pad_vals(adj_values, 128)

    zc = _tc_k1(feat, feat_a, weight1)
    pa = _sc_spmm(zc, pack_a128, vals_a128, scaled=True, nbuf=2)
    hiden_emb, emb64, emb128, dec, dec_a = _tc_k2(
        n, pa, dec_w1.T, dec_b1.reshape(1, -1), dec_w2.T, dec_b2.reshape(1, -1))
    pb = _sc_spmm(emb64, pack_a128, vals_a128, scaled=True, nbuf=5)
    pc = _sc_spmm(emb128, pack_n128, None, scaled=False, nbuf=2)
    h, ret, ret_a = _tc_k3(n, pb, pc, weight2)
    return (hiden_emb, h, dec, dec_a, ret, ret_a)
